# Initial kernel scaffold; baseline (speedup 1.0000x reference)
import jax
import jax.numpy as jnp
from jax.experimental import pallas as pl


def kernel(embs, h_gs, rel_embs, rel_transe_embs, params):
    raise NotImplementedError("write your pallas kernel here")


# speedup vs baseline: 1.5955x; 1.5955x over previous
import jax
import jax.numpy as jnp
from jax.experimental import pallas as pl
from jax.experimental.pallas import tpu as pltpu

H = 768
TE = 100
R = 16
B = 4
L = 512
TH = 0.5


def _brask_kernel(embs_ref, hgs_ref, relte_ref, rele_ref, rproj_ref,
                  Ws_ref, Wr_ref, Wg_ref, Wx_ref, Wx2_ref,
                  vecs_ref, bias_ref, scal_ref, start_ref, end_ref):
    f32 = jnp.float32
    embs_b = embs_ref[0]
    hgs_b = hgs_ref[0]
    bias = bias_ref[0]
    vecs = vecs_ref[0]

    def mm(a, b):
        return jnp.dot(a, b, preferred_element_type=f32)

    ev = mm(embs_b, vecs)

    sp = jax.nn.sigmoid(ev[:, 0:1] + scal_ref[0, 0, 0])
    ep = jax.nn.sigmoid(ev[:, 1:2] + scal_ref[0, 0, 1])
    ws = sp * (sp > TH)
    we = ep * (ep > TH)
    num_s = jnp.sum(ws * embs_b, axis=0, keepdims=True)
    num_e = jnp.sum(we * embs_b, axis=0, keepdims=True)
    den_s = jnp.sum(ws, axis=0, keepdims=True) + 1e-6
    den_e = jnp.sum(we, axis=0, keepdims=True) + 1e-6
    span = 0.5 * (num_s / den_s + num_e / den_e)

    h_d = embs_b + (mm(span, Ws_ref[0]) + bias[0:1, :])

    r_h = mm(relte_ref[...], rproj_ref[...]) + bias[5:6, :] + rele_ref[...]
    rW = mm(r_h, Wr_ref[0]) + bias[1:2, :]

    g = mm(embs_b, Wg_ref[0]) + bias[2:3, :]
    x = mm(hgs_b, Wx_ref[0]) + bias[3:4, :]
    gx = g + x
    vcol = vecs[:, 4:5]
    cols = []
    for r in range(R):
        t = jnp.tanh(gx + rW[r:r + 1, :])
        cols.append(mm(t, vcol))
    e = jnp.concatenate(cols, axis=1) + scal_ref[0, 0, 4]

    m = jnp.max(e, axis=0, keepdims=True)
    a = jnp.exp(e - m)
    a = a / jnp.sum(a, axis=0, keepdims=True)

    h2 = jnp.tanh(mm(h_d, Wx2_ref[0]) + bias[4:5, :])

    hv = mm(h2, vecs)
    hw_s = hv[:, 2:3] + scal_ref[0, 0, 2]
    hw_e = hv[:, 3:4] + scal_ref[0, 0, 3]
    cw_s = jnp.sum(ev[:, 2:3] * a, axis=0, keepdims=True)
    cw_e = jnp.sum(ev[:, 3:4] * a, axis=0, keepdims=True)
    start_ref[0, 0] = jax.nn.sigmoid(hw_s + cw_s)
    end_ref[0, 0] = jax.nn.sigmoid(hw_e + cw_e)


def kernel(embs, h_gs, rel_embs, rel_transe_embs, params):
    p = params
    f32 = jnp.float32

    def stk(a, b):
        return jnp.stack([a, b], axis=0)

    Ws = stk(p['f_W_s_W'], p['b_W_s_W'])
    Wr = stk(p['f_W_r_W'], p['b_W_r_W'])
    Wg = stk(p['f_W_g_W'], p['b_W_g_W'])
    Wx = stk(p['f_W_x_W'], p['b_W_x_W'])
    Wx2 = stk(p['f_Wx2_W'], p['b_Wx2_W'])

    def vpack(tag_s, tag_e, head_s, head_e):
        return jnp.concatenate(
            [p[tag_s + '_W'], p[tag_e + '_W'], p[head_s + '_W'],
             p[head_e + '_W'], p['V_W'], jnp.zeros((H, 3), f32)], axis=1)

    vecs = stk(vpack('f_start_sub_fc', 'f_end_sub_fc',
                     'f_start_obj_fc', 'f_end_obj_fc'),
               vpack('b_start_obj_fc', 'b_end_obj_fc',
                     'b_start_sub_fc', 'b_end_sub_fc'))

    def scalars(tag_s, tag_e, head_s, head_e):
        return jnp.concatenate([
            p[tag_s + '_b'], p[tag_e + '_b'], p[head_s + '_b'],
            p[head_e + '_b'], p['V_b'], jnp.zeros((3,), f32)])

    scal = stk(scalars('f_start_sub_fc', 'f_end_sub_fc',
                       'f_start_obj_fc', 'f_end_obj_fc'),
               scalars('b_start_obj_fc', 'b_end_obj_fc',
                       'b_start_sub_fc', 'b_end_sub_fc'))

    def biaspack(pre):
        rows = [p[pre + '_W_s_b'], p[pre + '_W_r_b'], p[pre + '_W_g_b'],
                p[pre + '_W_x_b'], p[pre + '_Wx2_b'],
                p['r_proj_b'], jnp.zeros((H,), f32), jnp.zeros((H,), f32)]
        return jnp.stack(rows, axis=0)

    bias = stk(biaspack('f'), biaspack('b'))

    grid = (2, B)
    start, end = pl.pallas_call(
        _brask_kernel,
        grid=grid,
        in_specs=[
            pl.BlockSpec((1, L, H), lambda d, b: (b, 0, 0)),
            pl.BlockSpec((1, 1, H), lambda d, b: (b, 0, 0)),
            pl.BlockSpec((R, TE), lambda d, b: (0, 0)),
            pl.BlockSpec((R, H), lambda d, b: (0, 0)),
            pl.BlockSpec((TE, H), lambda d, b: (0, 0)),
            pl.BlockSpec((1, H, H), lambda d, b: (d, 0, 0)),
            pl.BlockSpec((1, H, H), lambda d, b: (d, 0, 0)),
            pl.BlockSpec((1, H, H), lambda d, b: (d, 0, 0)),
            pl.BlockSpec((1, H, H), lambda d, b: (d, 0, 0)),
            pl.BlockSpec((1, H, H), lambda d, b: (d, 0, 0)),
            pl.BlockSpec((1, H, 8), lambda d, b: (d, 0, 0)),
            pl.BlockSpec((1, 8, H), lambda d, b: (d, 0, 0)),
            pl.BlockSpec((1, 1, 8), lambda d, b: (d, 0, 0),
                         memory_space=pltpu.SMEM),
        ],
        out_specs=[
            pl.BlockSpec((1, 1, L, R), lambda d, b: (d, b, 0, 0)),
            pl.BlockSpec((1, 1, L, R), lambda d, b: (d, b, 0, 0)),
        ],
        out_shape=[
            jax.ShapeDtypeStruct((2, B, L, R), f32),
            jax.ShapeDtypeStruct((2, B, L, R), f32),
        ],
        compiler_params=pltpu.CompilerParams(
            dimension_semantics=("parallel", "parallel")),
    )(embs, h_gs.reshape(B, 1, H), rel_transe_embs, rel_embs, p['r_proj_W'],
      Ws, Wr, Wg, Wx, Wx2, vecs, bias, scal.reshape(2, 1, 8))

    return jnp.stack([start[0], end[0], start[1], end[1]], axis=0)


# speedup vs baseline: 1.6490x; 1.0335x over previous
import jax
import jax.numpy as jnp
from jax.experimental import pallas as pl
from jax.experimental.pallas import tpu as pltpu

H = 768
TE = 100
R = 16
B = 4
L = 512
TH = 0.5


def _brask_kernel(embs_ref, hgs_ref, relte_ref, rele_ref, rproj_ref,
                  Ws_ref, Wr_ref, Wg_ref, Wx_ref, Wx2_ref,
                  vecs_ref, bias_ref, scal_ref, start_ref, end_ref):
    f32 = jnp.float32
    embs_b = embs_ref[0]
    hgs_b = hgs_ref[0]
    bias = bias_ref[0]
    vecs = vecs_ref[0]

    bf16 = jnp.bfloat16

    def mm(a, b):
        return jnp.dot(a, b, preferred_element_type=f32)

    embs_bf = embs_b.astype(bf16)

    ev = mm(embs_b, vecs)

    sp = jax.nn.sigmoid(ev[:, 0:1] + scal_ref[0, 0, 0])
    ep = jax.nn.sigmoid(ev[:, 1:2] + scal_ref[0, 0, 1])
    ws = sp * (sp > TH)
    we = ep * (ep > TH)
    num_s = jnp.sum(ws * embs_b, axis=0, keepdims=True)
    num_e = jnp.sum(we * embs_b, axis=0, keepdims=True)
    den_s = jnp.sum(ws, axis=0, keepdims=True) + 1e-6
    den_e = jnp.sum(we, axis=0, keepdims=True) + 1e-6
    span = 0.5 * (num_s / den_s + num_e / den_e)

    h_d = embs_b + (mm(span.astype(bf16), Ws_ref[0]) + bias[0:1, :])

    r_h = mm(relte_ref[...], rproj_ref[...]) + bias[5:6, :] + rele_ref[...]
    rW = mm(r_h.astype(bf16), Wr_ref[0]) + bias[1:2, :]

    g = mm(embs_bf, Wg_ref[0]) + bias[2:3, :]
    x = mm(hgs_b.astype(bf16), Wx_ref[0]) + bias[3:4, :]
    gx = (g + x).astype(bf16)
    rW_bf = rW.astype(bf16)
    vcol = vecs[:, 4:5].astype(bf16)
    cols = []
    for r in range(R):
        t = jnp.tanh(gx + rW_bf[r:r + 1, :])
        cols.append(mm(t, vcol))
    e = jnp.concatenate(cols, axis=1) + scal_ref[0, 0, 4]

    m = jnp.max(e, axis=0, keepdims=True)
    a = jnp.exp(e - m)
    a = a / jnp.sum(a, axis=0, keepdims=True)

    h2 = jnp.tanh(mm(h_d.astype(bf16), Wx2_ref[0]) + bias[4:5, :])

    hv = mm(h2.astype(bf16), vecs.astype(bf16))
    hw_s = hv[:, 2:3] + scal_ref[0, 0, 2]
    hw_e = hv[:, 3:4] + scal_ref[0, 0, 3]
    cw_s = jnp.sum(ev[:, 2:3] * a, axis=0, keepdims=True)
    cw_e = jnp.sum(ev[:, 3:4] * a, axis=0, keepdims=True)
    start_ref[0, 0] = jax.nn.sigmoid(hw_s + cw_s)
    end_ref[0, 0] = jax.nn.sigmoid(hw_e + cw_e)


def kernel(embs, h_gs, rel_embs, rel_transe_embs, params):
    p = params
    f32 = jnp.float32

    def stk(a, b):
        return jnp.stack([a, b], axis=0)

    bf16 = jnp.bfloat16
    Ws = stk(p['f_W_s_W'], p['b_W_s_W']).astype(bf16)
    Wr = stk(p['f_W_r_W'], p['b_W_r_W']).astype(bf16)
    Wg = stk(p['f_W_g_W'], p['b_W_g_W']).astype(bf16)
    Wx = stk(p['f_W_x_W'], p['b_W_x_W']).astype(bf16)
    Wx2 = stk(p['f_Wx2_W'], p['b_Wx2_W']).astype(bf16)

    def vpack(tag_s, tag_e, head_s, head_e):
        return jnp.concatenate(
            [p[tag_s + '_W'], p[tag_e + '_W'], p[head_s + '_W'],
             p[head_e + '_W'], p['V_W'], jnp.zeros((H, 3), f32)], axis=1)

    vecs = stk(vpack('f_start_sub_fc', 'f_end_sub_fc',
                     'f_start_obj_fc', 'f_end_obj_fc'),
               vpack('b_start_obj_fc', 'b_end_obj_fc',
                     'b_start_sub_fc', 'b_end_sub_fc'))

    def scalars(tag_s, tag_e, head_s, head_e):
        return jnp.concatenate([
            p[tag_s + '_b'], p[tag_e + '_b'], p[head_s + '_b'],
            p[head_e + '_b'], p['V_b'], jnp.zeros((3,), f32)])

    scal = stk(scalars('f_start_sub_fc', 'f_end_sub_fc',
                       'f_start_obj_fc', 'f_end_obj_fc'),
               scalars('b_start_obj_fc', 'b_end_obj_fc',
                       'b_start_sub_fc', 'b_end_sub_fc'))

    def biaspack(pre):
        rows = [p[pre + '_W_s_b'], p[pre + '_W_r_b'], p[pre + '_W_g_b'],
                p[pre + '_W_x_b'], p[pre + '_Wx2_b'],
                p['r_proj_b'], jnp.zeros((H,), f32), jnp.zeros((H,), f32)]
        return jnp.stack(rows, axis=0)

    bias = stk(biaspack('f'), biaspack('b'))

    grid = (2, B)
    start, end = pl.pallas_call(
        _brask_kernel,
        grid=grid,
        in_specs=[
            pl.BlockSpec((1, L, H), lambda d, b: (b, 0, 0)),
            pl.BlockSpec((1, 1, H), lambda d, b: (b, 0, 0)),
            pl.BlockSpec((R, TE), lambda d, b: (0, 0)),
            pl.BlockSpec((R, H), lambda d, b: (0, 0)),
            pl.BlockSpec((TE, H), lambda d, b: (0, 0)),
            pl.BlockSpec((1, H, H), lambda d, b: (d, 0, 0)),
            pl.BlockSpec((1, H, H), lambda d, b: (d, 0, 0)),
            pl.BlockSpec((1, H, H), lambda d, b: (d, 0, 0)),
            pl.BlockSpec((1, H, H), lambda d, b: (d, 0, 0)),
            pl.BlockSpec((1, H, H), lambda d, b: (d, 0, 0)),
            pl.BlockSpec((1, H, 8), lambda d, b: (d, 0, 0)),
            pl.BlockSpec((1, 8, H), lambda d, b: (d, 0, 0)),
            pl.BlockSpec((1, 1, 8), lambda d, b: (d, 0, 0),
                         memory_space=pltpu.SMEM),
        ],
        out_specs=[
            pl.BlockSpec((1, 1, L, R), lambda d, b: (d, b, 0, 0)),
            pl.BlockSpec((1, 1, L, R), lambda d, b: (d, b, 0, 0)),
        ],
        out_shape=[
            jax.ShapeDtypeStruct((2, B, L, R), f32),
            jax.ShapeDtypeStruct((2, B, L, R), f32),
        ],
        compiler_params=pltpu.CompilerParams(
            dimension_semantics=("parallel", "parallel")),
    )(embs, h_gs.reshape(B, 1, H), rel_transe_embs, rel_embs, p['r_proj_W'],
      Ws, Wr, Wg, Wx, Wx2, vecs, bias, scal.reshape(2, 1, 8))

    return jnp.stack([start[0], end[0], start[1], end[1]], axis=0)


# speedup vs baseline: 1.9076x; 1.1568x over previous
import jax
import jax.numpy as jnp
from jax.experimental import pallas as pl
from jax.experimental.pallas import tpu as pltpu

H = 768
TE = 100
R = 16
B = 4
L = 512
TH = 0.5


def _brask_kernel(embs_ref, hgs_ref, relte_ref, rele_ref, rproj_ref,
                  fWs_ref, fWr_ref, fWg_ref, fWx_ref, fWx2_ref,
                  bWs_ref, bWr_ref, bWg_ref, bWx_ref, bWx2_ref,
                  vecs_ref, bias_ref, scal_ref,
                  fos_ref, foe_ref, bss_ref, bse_ref):
    f32 = jnp.float32
    embs_b = embs_ref[0]
    hgs_b = hgs_ref[0]

    def mm(a, b):
        return jnp.dot(a, b, preferred_element_type=f32)

    r_h = (mm(relte_ref[...], rproj_ref[...])
           + bias_ref[0, 5:6, :] + rele_ref[...])

    def direction(d, Ws, Wr, Wg, Wx, Wx2, out_s_ref, out_e_ref):
        vecs = vecs_ref[d]
        bias = bias_ref[d]

        ev = mm(embs_b, vecs)

        sp = jax.nn.sigmoid(ev[:, 0:1] + scal_ref[d, 0, 0])
        ep = jax.nn.sigmoid(ev[:, 1:2] + scal_ref[d, 0, 1])
        ws = sp * (sp > TH)
        we = ep * (ep > TH)
        num_s = jnp.sum(ws * embs_b, axis=0, keepdims=True)
        num_e = jnp.sum(we * embs_b, axis=0, keepdims=True)
        den_s = jnp.sum(ws, axis=0, keepdims=True) + 1e-6
        den_e = jnp.sum(we, axis=0, keepdims=True) + 1e-6
        span = 0.5 * (num_s / den_s + num_e / den_e)

        h_d = embs_b + (mm(span, Ws) + bias[0:1, :])

        rW = mm(r_h, Wr) + bias[1:2, :]

        g = mm(embs_b, Wg) + bias[2:3, :]
        x = mm(hgs_b, Wx) + bias[3:4, :]
        gx = g + x
        vcol = vecs[:, 4:5]
        cols = []
        for r in range(R):
            t = jnp.tanh(gx + rW[r:r + 1, :])
            cols.append(mm(t, vcol))
        e = jnp.concatenate(cols, axis=1) + scal_ref[d, 0, 4]

        m = jnp.max(e, axis=0, keepdims=True)
        a = jnp.exp(e - m)
        a = a / jnp.sum(a, axis=0, keepdims=True)

        h2 = jnp.tanh(mm(h_d, Wx2) + bias[4:5, :])

        hv = mm(h2, vecs)
        hw_s = hv[:, 2:3] + scal_ref[d, 0, 2]
        hw_e = hv[:, 3:4] + scal_ref[d, 0, 3]
        cw_s = jnp.sum(ev[:, 2:3] * a, axis=0, keepdims=True)
        cw_e = jnp.sum(ev[:, 3:4] * a, axis=0, keepdims=True)
        out_s_ref[0] = jax.nn.sigmoid(hw_s + cw_s)
        out_e_ref[0] = jax.nn.sigmoid(hw_e + cw_e)

    direction(0, fWs_ref[...], fWr_ref[...], fWg_ref[...], fWx_ref[...],
              fWx2_ref[...], fos_ref, foe_ref)
    direction(1, bWs_ref[...], bWr_ref[...], bWg_ref[...], bWx_ref[...],
              bWx2_ref[...], bss_ref, bse_ref)


def kernel(embs, h_gs, rel_embs, rel_transe_embs, params):
    p = params
    f32 = jnp.float32

    def stk(a, b):
        return jnp.stack([a, b], axis=0)

    def vpack(tag_s, tag_e, head_s, head_e):
        return jnp.concatenate(
            [p[tag_s + '_W'], p[tag_e + '_W'], p[head_s + '_W'],
             p[head_e + '_W'], p['V_W'], jnp.zeros((H, 3), f32)], axis=1)

    vecs = stk(vpack('f_start_sub_fc', 'f_end_sub_fc',
                     'f_start_obj_fc', 'f_end_obj_fc'),
               vpack('b_start_obj_fc', 'b_end_obj_fc',
                     'b_start_sub_fc', 'b_end_sub_fc'))

    def scalars(tag_s, tag_e, head_s, head_e):
        return jnp.concatenate([
            p[tag_s + '_b'], p[tag_e + '_b'], p[head_s + '_b'],
            p[head_e + '_b'], p['V_b'], jnp.zeros((3,), f32)])

    scal = stk(scalars('f_start_sub_fc', 'f_end_sub_fc',
                       'f_start_obj_fc', 'f_end_obj_fc'),
               scalars('b_start_obj_fc', 'b_end_obj_fc',
                       'b_start_sub_fc', 'b_end_sub_fc'))

    def biaspack(pre):
        rows = [p[pre + '_W_s_b'], p[pre + '_W_r_b'], p[pre + '_W_g_b'],
                p[pre + '_W_x_b'], p[pre + '_Wx2_b'],
                p['r_proj_b'], jnp.zeros((H,), f32), jnp.zeros((H,), f32)]
        return jnp.stack(rows, axis=0)

    bias = stk(biaspack('f'), biaspack('b'))

    full2 = pl.BlockSpec((R, TE), lambda b: (0, 0))
    W_spec = pl.BlockSpec((H, H), lambda b: (0, 0))
    out_spec = pl.BlockSpec((1, L, R), lambda b: (b, 0, 0))
    outs = pl.pallas_call(
        _brask_kernel,
        grid=(B,),
        in_specs=[
            pl.BlockSpec((1, L, H), lambda b: (b, 0, 0)),
            pl.BlockSpec((1, 1, H), lambda b: (b, 0, 0)),
            full2,
            pl.BlockSpec((R, H), lambda b: (0, 0)),
            pl.BlockSpec((TE, H), lambda b: (0, 0)),
            W_spec, W_spec, W_spec, W_spec, W_spec,
            W_spec, W_spec, W_spec, W_spec, W_spec,
            pl.BlockSpec((2, H, 8), lambda b: (0, 0, 0)),
            pl.BlockSpec((2, 8, H), lambda b: (0, 0, 0)),
            pl.BlockSpec((2, 1, 8), lambda b: (0, 0, 0),
                         memory_space=pltpu.SMEM),
        ],
        out_specs=[out_spec, out_spec, out_spec, out_spec],
        out_shape=[jax.ShapeDtypeStruct((B, L, R), f32)] * 4,
        compiler_params=pltpu.CompilerParams(
            dimension_semantics=("parallel",)),
    )(embs, h_gs.reshape(B, 1, H), rel_transe_embs, rel_embs, p['r_proj_W'],
      p['f_W_s_W'], p['f_W_r_W'], p['f_W_g_W'], p['f_W_x_W'], p['f_Wx2_W'],
      p['b_W_s_W'], p['b_W_r_W'], p['b_W_g_W'], p['b_W_x_W'], p['b_Wx2_W'],
      vecs, bias, scal.reshape(2, 1, 8))

    return jnp.stack(outs, axis=0)


# speedup vs baseline: 1.9971x; 1.0469x over previous
import jax
import jax.numpy as jnp
from jax.experimental import pallas as pl
from jax.experimental.pallas import tpu as pltpu

H = 768
TE = 100
R = 16
B = 4
L = 512
TH = 0.5


BT = 2


def _brask_kernel(embs_ref, hgs_ref, relte_ref, rele_ref, rproj_ref,
                  fWs_ref, fWr_ref, fWg_ref, fWx_ref, fWx2_ref,
                  bWs_ref, bWr_ref, bWg_ref, bWx_ref, bWx2_ref,
                  vecs_ref, bias_ref, scal_ref,
                  fos_ref, foe_ref, bss_ref, bse_ref):
    f32 = jnp.float32
    bf16 = jnp.bfloat16
    embs3 = embs_ref[...]
    embs2 = embs3.reshape(BT * L, H)
    hgs = hgs_ref[...].reshape(BT, H)

    def mm(a, b):
        return jnp.dot(a, b, preferred_element_type=f32)

    r_h = (mm(relte_ref[...], rproj_ref[...])
           + bias_ref[0, 5:6, :] + rele_ref[...])

    def direction(d, Ws_ref, Wr_ref, Wg_ref, Wx_ref, Wx2_ref,
                  out_s_ref, out_e_ref):
        vecs = vecs_ref[d]
        bias = bias_ref[d]

        ev = mm(embs2, vecs).reshape(BT, L, 8)

        sp = jax.nn.sigmoid(ev[:, :, 0:1] + scal_ref[d, 0])
        ep = jax.nn.sigmoid(ev[:, :, 1:2] + scal_ref[d, 1])
        ws = sp * (sp > TH)
        we = ep * (ep > TH)
        num_s = jnp.sum(ws * embs3, axis=1, keepdims=True)
        num_e = jnp.sum(we * embs3, axis=1, keepdims=True)
        den_s = jnp.sum(ws, axis=1, keepdims=True) + 1e-6
        den_e = jnp.sum(we, axis=1, keepdims=True) + 1e-6
        span = 0.5 * (num_s / den_s + num_e / den_e)

        spanW = mm(span.reshape(BT, H), Ws_ref[...]) + bias[0:1, :]
        h_d = (embs3 + spanW.reshape(BT, 1, H)).astype(bf16)

        rW = mm(r_h, Wr_ref[...]) + bias[1:2, :]
        rW_bf = rW.astype(bf16)

        Wg_bf = Wg_ref[...].astype(bf16)
        g = mm(embs2.astype(bf16), Wg_bf).reshape(BT, L, H) + bias[2:3, :]
        x = mm(hgs.astype(bf16), Wx_ref[...].astype(bf16)) + bias[3:4, :]
        gx = (g + x.reshape(BT, 1, H)).astype(bf16)
        vcol = vecs[:, 4:5].astype(bf16)
        cols = []
        for r in range(R):
            t = jnp.tanh(gx + rW_bf[r:r + 1, :])
            cols.append(mm(t.reshape(BT * L, H), vcol))
        e = (jnp.concatenate(cols, axis=1).reshape(BT, L, R)
             + scal_ref[d, 4])

        m = jnp.max(e, axis=1, keepdims=True)
        a = jnp.exp(e - m)
        a = a / jnp.sum(a, axis=1, keepdims=True)

        Wx2_bf = Wx2_ref[...].astype(bf16)
        h2 = jnp.tanh(mm(h_d.reshape(BT * L, H), Wx2_bf)
                      .reshape(BT, L, H) + bias[4:5, :])

        hv = mm(h2.reshape(BT * L, H).astype(bf16),
                vecs.astype(bf16)).reshape(BT, L, 8)
        hw_s = hv[:, :, 2:3] + scal_ref[d, 2]
        hw_e = hv[:, :, 3:4] + scal_ref[d, 3]
        cw_s = jnp.sum(ev[:, :, 2:3] * a, axis=1, keepdims=True)
        cw_e = jnp.sum(ev[:, :, 3:4] * a, axis=1, keepdims=True)
        out_s_ref[...] = jax.nn.sigmoid(hw_s + cw_s)
        out_e_ref[...] = jax.nn.sigmoid(hw_e + cw_e)

    direction(0, fWs_ref, fWr_ref, fWg_ref, fWx_ref, fWx2_ref,
              fos_ref, foe_ref)
    direction(1, bWs_ref, bWr_ref, bWg_ref, bWx_ref, bWx2_ref,
              bss_ref, bse_ref)


def kernel(embs, h_gs, rel_embs, rel_transe_embs, params):
    p = params
    f32 = jnp.float32

    def stk(a, b):
        return jnp.stack([a, b], axis=0)

    def vpack(tag_s, tag_e, head_s, head_e):
        return jnp.concatenate(
            [p[tag_s + '_W'], p[tag_e + '_W'], p[head_s + '_W'],
             p[head_e + '_W'], p['V_W'], jnp.zeros((H, 3), f32)], axis=1)

    vecs = stk(vpack('f_start_sub_fc', 'f_end_sub_fc',
                     'f_start_obj_fc', 'f_end_obj_fc'),
               vpack('b_start_obj_fc', 'b_end_obj_fc',
                     'b_start_sub_fc', 'b_end_sub_fc'))

    def scalars(tag_s, tag_e, head_s, head_e):
        return jnp.concatenate([
            p[tag_s + '_b'], p[tag_e + '_b'], p[head_s + '_b'],
            p[head_e + '_b'], p['V_b'], jnp.zeros((3,), f32)])

    scal = stk(scalars('f_start_sub_fc', 'f_end_sub_fc',
                       'f_start_obj_fc', 'f_end_obj_fc'),
               scalars('b_start_obj_fc', 'b_end_obj_fc',
                       'b_start_sub_fc', 'b_end_sub_fc'))

    def biaspack(pre):
        rows = [p[pre + '_W_s_b'], p[pre + '_W_r_b'], p[pre + '_W_g_b'],
                p[pre + '_W_x_b'], p[pre + '_Wx2_b'],
                p['r_proj_b'], jnp.zeros((H,), f32), jnp.zeros((H,), f32)]
        return jnp.stack(rows, axis=0)

    bias = stk(biaspack('f'), biaspack('b'))

    whole = pl.BlockSpec(memory_space=pltpu.VMEM)
    out_spec = pl.BlockSpec((BT, L, R), lambda b: (b, 0, 0))
    outs = pl.pallas_call(
        _brask_kernel,
        grid=(B // BT,),
        in_specs=[pl.BlockSpec((BT, L, H), lambda b: (b, 0, 0)),
                  pl.BlockSpec((BT, 1, H), lambda b: (b, 0, 0))]
        + [whole] * 15
        + [pl.BlockSpec(memory_space=pltpu.SMEM)],
        out_specs=[out_spec] * 4,
        out_shape=[jax.ShapeDtypeStruct((B, L, R), f32)] * 4,
        compiler_params=pltpu.CompilerParams(
            dimension_semantics=("arbitrary",)),
    )(embs, h_gs.reshape(B, 1, H), rel_transe_embs, rel_embs, p['r_proj_W'],
      p['f_W_s_W'], p['f_W_r_W'], p['f_W_g_W'], p['f_W_x_W'], p['f_Wx2_W'],
      p['b_W_s_W'], p['b_W_r_W'], p['b_W_g_W'], p['b_W_x_W'], p['b_Wx2_W'],
      vecs, bias, scal)

    return jnp.stack(outs, axis=0)


# speedup vs baseline: 2.0332x; 1.0181x over previous
import jax
import jax.numpy as jnp
from jax.experimental import pallas as pl
from jax.experimental.pallas import tpu as pltpu

H = 768
TE = 100
R = 16
B = 4
L = 512
TH = 0.5
BT = 2


def _brask_kernel(embs_ref, hgs_ref, relte_ref, rele_ref, rproj_ref,
                  fWs_ref, fWr_ref, fWg_ref, fWx_ref, fWx2_ref,
                  bWs_ref, bWr_ref, bWg_ref, bWx_ref, bWx2_ref,
                  vecs_ref, bias_ref, scal_ref, out_ref):
    f32 = jnp.float32
    bf16 = jnp.bfloat16
    embs3 = embs_ref[...]
    embs2 = embs3.reshape(BT * L, H)
    embs2_bf = embs2.astype(bf16)
    hgs_bf = hgs_ref[...].reshape(BT, H).astype(bf16)

    def mm(a, b):
        return jnp.dot(a, b, preferred_element_type=f32)

    r_h = (mm(relte_ref[...], rproj_ref[...])
           + bias_ref[0, 5:6, :] + rele_ref[...])
    r_h_bf = r_h.astype(bf16)

    vec_f = vecs_ref[0]
    vec_b = vecs_ref[1]
    bias_f = bias_ref[0]
    bias_b = bias_ref[1]

    ev_f = mm(embs2, vec_f).reshape(BT, L, 8)
    ev_b = mm(embs2, vec_b).reshape(BT, L, 8)

    def span_emb(ev, d):
        sp = jax.nn.sigmoid(ev[:, :, 0:1] + scal_ref[d, 0])
        ep = jax.nn.sigmoid(ev[:, :, 1:2] + scal_ref[d, 1])
        ws = sp * (sp > TH)
        we = ep * (ep > TH)
        num_s = jnp.sum(ws * embs3, axis=1, keepdims=True)
        num_e = jnp.sum(we * embs3, axis=1, keepdims=True)
        den_s = jnp.sum(ws, axis=1, keepdims=True) + 1e-6
        den_e = jnp.sum(we, axis=1, keepdims=True) + 1e-6
        return 0.5 * (num_s / den_s + num_e / den_e)

    span_f = span_emb(ev_f, 0)
    span_b = span_emb(ev_b, 1)
    spanW_f = mm(span_f.reshape(BT, H), fWs_ref[...]) + bias_f[0:1, :]
    spanW_b = mm(span_b.reshape(BT, H), bWs_ref[...]) + bias_b[0:1, :]
    hd_f = (embs3 + spanW_f.reshape(BT, 1, H)).astype(bf16)
    hd_b = (embs3 + spanW_b.reshape(BT, 1, H)).astype(bf16)

    rW_f = (mm(r_h_bf, fWr_ref[...].astype(bf16))
            + bias_f[1:2, :]).astype(bf16)
    rW_b = (mm(r_h_bf, bWr_ref[...].astype(bf16))
            + bias_b[1:2, :]).astype(bf16)
    g_f = mm(embs2_bf, fWg_ref[...].astype(bf16)).reshape(BT, L, H)
    g_b = mm(embs2_bf, bWg_ref[...].astype(bf16)).reshape(BT, L, H)
    x_f = mm(hgs_bf, fWx_ref[...].astype(bf16)) + bias_f[3:4, :]
    x_b = mm(hgs_bf, bWx_ref[...].astype(bf16)) + bias_b[3:4, :]
    gx_f = (g_f + bias_f[2:3, :] + x_f.reshape(BT, 1, H)).astype(bf16)
    gx_b = (g_b + bias_b[2:3, :] + x_b.reshape(BT, 1, H)).astype(bf16)

    vcol_f = vec_f[:, 4:5].astype(bf16)
    vcol_b = vec_b[:, 4:5].astype(bf16)
    cols_f, cols_b = [], []
    for r in range(R):
        t_f = jnp.tanh(gx_f + rW_f[r:r + 1, :])
        t_b = jnp.tanh(gx_b + rW_b[r:r + 1, :])
        cols_f.append(mm(t_f.reshape(BT * L, H), vcol_f))
        cols_b.append(mm(t_b.reshape(BT * L, H), vcol_b))
    e_f = (jnp.concatenate(cols_f, axis=1).reshape(BT, L, R)
           + scal_ref[0, 4])
    e_b = (jnp.concatenate(cols_b, axis=1).reshape(BT, L, R)
           + scal_ref[1, 4])

    def softmax_l(e):
        m = jnp.max(e, axis=1, keepdims=True)
        a = jnp.exp(e - m)
        return a / jnp.sum(a, axis=1, keepdims=True)

    a_f = softmax_l(e_f)
    a_b = softmax_l(e_b)

    h2_f = jnp.tanh(mm(hd_f.reshape(BT * L, H), fWx2_ref[...].astype(bf16))
                    .reshape(BT, L, H) + bias_f[4:5, :])
    h2_b = jnp.tanh(mm(hd_b.reshape(BT * L, H), bWx2_ref[...].astype(bf16))
                    .reshape(BT, L, H) + bias_b[4:5, :])

    hv_f = mm(h2_f.reshape(BT * L, H).astype(bf16),
              vec_f.astype(bf16)).reshape(BT, L, 8)
    hv_b = mm(h2_b.reshape(BT * L, H).astype(bf16),
              vec_b.astype(bf16)).reshape(BT, L, 8)

    def heads(hv, ev, a, d):
        hw_s = hv[:, :, 2:3] + scal_ref[d, 2]
        hw_e = hv[:, :, 3:4] + scal_ref[d, 3]
        cw_s = jnp.sum(ev[:, :, 2:3] * a, axis=1, keepdims=True)
        cw_e = jnp.sum(ev[:, :, 3:4] * a, axis=1, keepdims=True)
        return jax.nn.sigmoid(hw_s + cw_s), jax.nn.sigmoid(hw_e + cw_e)

    fos, foe = heads(hv_f, ev_f, a_f, 0)
    bss, bse = heads(hv_b, ev_b, a_b, 1)
    out_ref[0] = fos
    out_ref[1] = foe
    out_ref[2] = bss
    out_ref[3] = bse


def kernel(embs, h_gs, rel_embs, rel_transe_embs, params):
    p = params
    f32 = jnp.float32

    def stk(a, b):
        return jnp.stack([a, b], axis=0)

    def vpack(tag_s, tag_e, head_s, head_e):
        return jnp.concatenate(
            [p[tag_s + '_W'], p[tag_e + '_W'], p[head_s + '_W'],
             p[head_e + '_W'], p['V_W'], jnp.zeros((H, 3), f32)], axis=1)

    vecs = stk(vpack('f_start_sub_fc', 'f_end_sub_fc',
                     'f_start_obj_fc', 'f_end_obj_fc'),
               vpack('b_start_obj_fc', 'b_end_obj_fc',
                     'b_start_sub_fc', 'b_end_sub_fc'))

    def scalars(tag_s, tag_e, head_s, head_e):
        return jnp.concatenate([
            p[tag_s + '_b'], p[tag_e + '_b'], p[head_s + '_b'],
            p[head_e + '_b'], p['V_b'], jnp.zeros((3,), f32)])

    scal = stk(scalars('f_start_sub_fc', 'f_end_sub_fc',
                       'f_start_obj_fc', 'f_end_obj_fc'),
               scalars('b_start_obj_fc', 'b_end_obj_fc',
                       'b_start_sub_fc', 'b_end_sub_fc'))

    def biaspack(pre):
        rows = [p[pre + '_W_s_b'], p[pre + '_W_r_b'], p[pre + '_W_g_b'],
                p[pre + '_W_x_b'], p[pre + '_Wx2_b'],
                p['r_proj_b'], jnp.zeros((H,), f32), jnp.zeros((H,), f32)]
        return jnp.stack(rows, axis=0)

    bias = stk(biaspack('f'), biaspack('b'))

    whole = pl.BlockSpec(memory_space=pltpu.VMEM)
    out = pl.pallas_call(
        _brask_kernel,
        grid=(B // BT,),
        in_specs=[pl.BlockSpec((BT, L, H), lambda b: (b, 0, 0)),
                  pl.BlockSpec((BT, 1, H), lambda b: (b, 0, 0))]
        + [whole] * 15
        + [pl.BlockSpec(memory_space=pltpu.SMEM)],
        out_specs=pl.BlockSpec((4, BT, L, R), lambda b: (0, b, 0, 0)),
        out_shape=jax.ShapeDtypeStruct((4, B, L, R), f32),
        compiler_params=pltpu.CompilerParams(
            dimension_semantics=("arbitrary",)),
    )(embs, h_gs.reshape(B, 1, H), rel_transe_embs, rel_embs, p['r_proj_W'],
      p['f_W_s_W'], p['f_W_r_W'], p['f_W_g_W'], p['f_W_x_W'], p['f_Wx2_W'],
      p['b_W_s_W'], p['b_W_r_W'], p['b_W_g_W'], p['b_W_x_W'], p['b_Wx2_W'],
      vecs, bias, scal)

    return out


# speedup vs baseline: 2.0406x; 1.0036x over previous
import jax
import jax.numpy as jnp
from jax.experimental import pallas as pl
from jax.experimental.pallas import tpu as pltpu

H = 768
TE = 100
R = 16
B = 4
L = 512
TH = 0.5
BT = 2


def _brask_kernel(embs_ref, hgs_ref, relte_ref, rele_ref, rproj_ref,
                  fWs_ref, fWr_ref, fWg_ref, fWx_ref, fWx2_ref,
                  bWs_ref, bWr_ref, bWg_ref, bWx_ref, bWx2_ref,
                  vecs_ref, bias_ref, scal_ref, out_ref):
    f32 = jnp.float32
    bf16 = jnp.bfloat16
    embs3 = embs_ref[...]
    embs2 = embs3.reshape(BT * L, H)
    embs2_bf = embs2.astype(bf16)
    hgs_bf = hgs_ref[...].reshape(BT, H).astype(bf16)

    def mm(a, b):
        return jnp.dot(a, b, preferred_element_type=f32)

    r_h = (mm(relte_ref[...], rproj_ref[...])
           + bias_ref[0, 5:6, :] + rele_ref[...])
    r_h_bf = r_h.astype(bf16)

    vec_f = vecs_ref[0]
    vec_b = vecs_ref[1]
    bias_f = bias_ref[0]
    bias_b = bias_ref[1]

    ev_f = mm(embs2, vec_f).reshape(BT, L, 8)
    ev_b = mm(embs2, vec_b).reshape(BT, L, 8)

    def span_emb(ev, d):
        sp = jax.nn.sigmoid(ev[:, :, 0:1] + scal_ref[d, 0])
        ep = jax.nn.sigmoid(ev[:, :, 1:2] + scal_ref[d, 1])
        ws = sp * (sp > TH)
        we = ep * (ep > TH)
        num_s = jnp.sum(ws * embs3, axis=1, keepdims=True)
        num_e = jnp.sum(we * embs3, axis=1, keepdims=True)
        den_s = jnp.sum(ws, axis=1, keepdims=True) + 1e-6
        den_e = jnp.sum(we, axis=1, keepdims=True) + 1e-6
        return 0.5 * (num_s / den_s + num_e / den_e)

    span_f = span_emb(ev_f, 0)
    span_b = span_emb(ev_b, 1)
    spanW_f = mm(span_f.reshape(BT, H), fWs_ref[...]) + bias_f[0:1, :]
    spanW_b = mm(span_b.reshape(BT, H), bWs_ref[...]) + bias_b[0:1, :]
    hd_f = (embs3 + spanW_f.reshape(BT, 1, H)).astype(bf16)
    hd_b = (embs3 + spanW_b.reshape(BT, 1, H)).astype(bf16)

    rW_f = (mm(r_h_bf, fWr_ref[...].astype(bf16))
            + bias_f[1:2, :]).astype(bf16)
    rW_b = (mm(r_h_bf, bWr_ref[...].astype(bf16))
            + bias_b[1:2, :]).astype(bf16)
    g_f = mm(embs2_bf, fWg_ref[...].astype(bf16)).reshape(BT, L, H)
    g_b = mm(embs2_bf, bWg_ref[...].astype(bf16)).reshape(BT, L, H)
    x_f = mm(hgs_bf, fWx_ref[...].astype(bf16)) + bias_f[3:4, :]
    x_b = mm(hgs_bf, bWx_ref[...].astype(bf16)) + bias_b[3:4, :]
    gx_f = (g_f + bias_f[2:3, :] + x_f.reshape(BT, 1, H)).astype(bf16)
    gx_b = (g_b + bias_b[2:3, :] + x_b.reshape(BT, 1, H)).astype(bf16)

    vcol_f = vec_f[:, 4:5].astype(bf16)
    vcol_b = vec_b[:, 4:5].astype(bf16)
    cols_f, cols_b = [], []
    for r in range(R):
        t_f = jnp.tanh(gx_f + rW_f[r:r + 1, :])
        t_b = jnp.tanh(gx_b + rW_b[r:r + 1, :])
        cols_f.append(mm(t_f.reshape(BT * L, H), vcol_f))
        cols_b.append(mm(t_b.reshape(BT * L, H), vcol_b))
    e_f = (jnp.concatenate(cols_f, axis=1).reshape(BT, L, R)
           + scal_ref[0, 4])
    e_b = (jnp.concatenate(cols_b, axis=1).reshape(BT, L, R)
           + scal_ref[1, 4])

    def softmax_l(e):
        m = jnp.max(e, axis=1, keepdims=True)
        a = jnp.exp(e - m)
        return a / jnp.sum(a, axis=1, keepdims=True)

    a_f = softmax_l(e_f)
    a_b = softmax_l(e_b)

    h2_f = jnp.tanh(mm(hd_f.reshape(BT * L, H), fWx2_ref[...].astype(bf16))
                    .reshape(BT, L, H) + bias_f[4:5, :])
    h2_b = jnp.tanh(mm(hd_b.reshape(BT * L, H), bWx2_ref[...].astype(bf16))
                    .reshape(BT, L, H) + bias_b[4:5, :])

    hv_f = mm(h2_f.reshape(BT * L, H).astype(bf16),
              vec_f.astype(bf16)).reshape(BT, L, 8)
    hv_b = mm(h2_b.reshape(BT * L, H).astype(bf16),
              vec_b.astype(bf16)).reshape(BT, L, 8)

    def heads(hv, ev, a, d):
        hw_s = hv[:, :, 2:3] + scal_ref[d, 2]
        hw_e = hv[:, :, 3:4] + scal_ref[d, 3]
        cw_s = jnp.sum(ev[:, :, 2:3] * a, axis=1, keepdims=True)
        cw_e = jnp.sum(ev[:, :, 3:4] * a, axis=1, keepdims=True)
        return jax.nn.sigmoid(hw_s + cw_s), jax.nn.sigmoid(hw_e + cw_e)

    fos, foe = heads(hv_f, ev_f, a_f, 0)
    bss, bse = heads(hv_b, ev_b, a_b, 1)
    out_ref[0] = fos
    out_ref[1] = foe
    out_ref[2] = bss
    out_ref[3] = bse


def kernel(embs, h_gs, rel_embs, rel_transe_embs, params):
    p = params
    f32 = jnp.float32

    def stk(a, b):
        return jnp.stack([a, b], axis=0)

    def vpack(tag_s, tag_e, head_s, head_e):
        return jnp.concatenate(
            [p[tag_s + '_W'], p[tag_e + '_W'], p[head_s + '_W'],
             p[head_e + '_W'], p['V_W'], jnp.zeros((H, 3), f32)], axis=1)

    vecs = stk(vpack('f_start_sub_fc', 'f_end_sub_fc',
                     'f_start_obj_fc', 'f_end_obj_fc'),
               vpack('b_start_obj_fc', 'b_end_obj_fc',
                     'b_start_sub_fc', 'b_end_sub_fc'))

    def scalars(tag_s, tag_e, head_s, head_e):
        return jnp.concatenate([
            p[tag_s + '_b'], p[tag_e + '_b'], p[head_s + '_b'],
            p[head_e + '_b'], p['V_b'], jnp.zeros((3,), f32)])

    scal = stk(scalars('f_start_sub_fc', 'f_end_sub_fc',
                       'f_start_obj_fc', 'f_end_obj_fc'),
               scalars('b_start_obj_fc', 'b_end_obj_fc',
                       'b_start_sub_fc', 'b_end_sub_fc'))

    def biaspack(pre):
        rows = [p[pre + '_W_s_b'], p[pre + '_W_r_b'], p[pre + '_W_g_b'],
                p[pre + '_W_x_b'], p[pre + '_Wx2_b'],
                p['r_proj_b'], jnp.zeros((H,), f32), jnp.zeros((H,), f32)]
        return jnp.stack(rows, axis=0)

    bias = stk(biaspack('f'), biaspack('b'))

    whole = pl.BlockSpec(memory_space=pltpu.VMEM)
    out = pl.pallas_call(
        _brask_kernel,
        grid=(B // BT,),
        in_specs=[pl.BlockSpec((BT, L, H), lambda b: (b, 0, 0)),
                  pl.BlockSpec((BT, 1, H), lambda b: (b, 0, 0))]
        + [whole] * 15
        + [pl.BlockSpec(memory_space=pltpu.SMEM)],
        out_specs=pl.BlockSpec((4, BT, L, R), lambda b: (0, b, 0, 0)),
        out_shape=jax.ShapeDtypeStruct((4, B, L, R), f32),
        compiler_params=pltpu.CompilerParams(
            dimension_semantics=("parallel",)),
    )(embs, h_gs.reshape(B, 1, H), rel_transe_embs, rel_embs, p['r_proj_W'],
      p['f_W_s_W'], p['f_W_r_W'], p['f_W_g_W'], p['f_W_x_W'], p['f_Wx2_W'],
      p['b_W_s_W'], p['b_W_r_W'], p['b_W_g_W'], p['b_W_x_W'], p['b_Wx2_W'],
      vecs, bias, scal)

    return out


# speedup vs baseline: 2.1026x; 1.0304x over previous
import jax
import jax.numpy as jnp
from jax.experimental import pallas as pl
from jax.experimental.pallas import tpu as pltpu

H = 768
TE = 100
R = 16
B = 4
L = 512
TH = 0.5
BT = 2


def _brask_kernel(embs_ref, hgs_ref, relte_ref, rele_ref, rproj_ref,
                  fWs_ref, fWr_ref, fWg_ref, fWx_ref, fWx2_ref,
                  bWs_ref, bWr_ref, bWg_ref, bWx_ref, bWx2_ref,
                  vecs_ref, bias_ref, scal_ref, out_ref):
    f32 = jnp.float32
    bf16 = jnp.bfloat16
    embs3 = embs_ref[...]
    embs2 = embs3.reshape(BT * L, H)
    embs2_bf = embs2.astype(bf16)
    hgs_bf = hgs_ref[...].reshape(BT, H).astype(bf16)

    def mm(a, b):
        return jnp.dot(a, b, preferred_element_type=f32)

    r_h = (mm(relte_ref[...], rproj_ref[...])
           + bias_ref[0, 5:6, :] + rele_ref[...])
    r_h_bf = r_h.astype(bf16)

    vec_f = vecs_ref[0]
    vec_b = vecs_ref[1]
    bias_f = bias_ref[0]
    bias_b = bias_ref[1]

    ev_f = mm(embs2, vec_f).reshape(BT, L, 8)
    ev_b = mm(embs2, vec_b).reshape(BT, L, 8)

    def span_emb(ev, d):
        sp = jax.nn.sigmoid(ev[:, :, 0:1] + scal_ref[d, 0])
        ep = jax.nn.sigmoid(ev[:, :, 1:2] + scal_ref[d, 1])
        ws = sp * (sp > TH)
        we = ep * (ep > TH)
        num_s = jnp.sum(ws * embs3, axis=1, keepdims=True)
        num_e = jnp.sum(we * embs3, axis=1, keepdims=True)
        den_s = jnp.sum(ws, axis=1, keepdims=True) + 1e-6
        den_e = jnp.sum(we, axis=1, keepdims=True) + 1e-6
        return 0.5 * (num_s / den_s + num_e / den_e)

    span_f = span_emb(ev_f, 0)
    span_b = span_emb(ev_b, 1)
    spanW_f = mm(span_f.reshape(BT, H), fWs_ref[...]) + bias_f[0:1, :]
    spanW_b = mm(span_b.reshape(BT, H), bWs_ref[...]) + bias_b[0:1, :]
    hd_f = (embs3 + spanW_f.reshape(BT, 1, H)).astype(bf16)
    hd_b = (embs3 + spanW_b.reshape(BT, 1, H)).astype(bf16)

    rW_f = (mm(r_h_bf, fWr_ref[...].astype(bf16))
            + bias_f[1:2, :]).astype(bf16)
    rW_b = (mm(r_h_bf, bWr_ref[...].astype(bf16))
            + bias_b[1:2, :]).astype(bf16)
    g_f = mm(embs2_bf, fWg_ref[...].astype(bf16)).reshape(BT, L, H)
    g_b = mm(embs2_bf, bWg_ref[...].astype(bf16)).reshape(BT, L, H)
    x_f = mm(hgs_bf, fWx_ref[...].astype(bf16)) + bias_f[3:4, :]
    x_b = mm(hgs_bf, bWx_ref[...].astype(bf16)) + bias_b[3:4, :]
    gx_f = (g_f + bias_f[2:3, :] + x_f.reshape(BT, 1, H)).astype(bf16)
    gx_b = (g_b + bias_b[2:3, :] + x_b.reshape(BT, 1, H)).astype(bf16)

    vrow = bias_f[6:7, :].reshape(1, 1, H).astype(bf16)
    vrow_f = vrow
    vrow_b = vrow
    cols_f, cols_b = [], []
    for r in range(R):
        t_f = jnp.tanh(gx_f + rW_f[r:r + 1, :])
        t_b = jnp.tanh(gx_b + rW_b[r:r + 1, :])
        cols_f.append(jnp.sum((t_f * vrow_f).astype(f32), axis=2,
                              keepdims=True))
        cols_b.append(jnp.sum((t_b * vrow_b).astype(f32), axis=2,
                              keepdims=True))
    e_f = jnp.concatenate(cols_f, axis=2) + scal_ref[0, 4]
    e_b = jnp.concatenate(cols_b, axis=2) + scal_ref[1, 4]

    def softmax_l(e):
        m = jnp.max(e, axis=1, keepdims=True)
        a = jnp.exp(e - m)
        return a / jnp.sum(a, axis=1, keepdims=True)

    a_f = softmax_l(e_f)
    a_b = softmax_l(e_b)

    h2_f = jnp.tanh(mm(hd_f.reshape(BT * L, H), fWx2_ref[...].astype(bf16))
                    .reshape(BT, L, H) + bias_f[4:5, :])
    h2_b = jnp.tanh(mm(hd_b.reshape(BT * L, H), bWx2_ref[...].astype(bf16))
                    .reshape(BT, L, H) + bias_b[4:5, :])

    hv_f = mm(h2_f.reshape(BT * L, H).astype(bf16),
              vec_f.astype(bf16)).reshape(BT, L, 8)
    hv_b = mm(h2_b.reshape(BT * L, H).astype(bf16),
              vec_b.astype(bf16)).reshape(BT, L, 8)

    def heads(hv, ev, a, d):
        hw_s = hv[:, :, 2:3] + scal_ref[d, 2]
        hw_e = hv[:, :, 3:4] + scal_ref[d, 3]
        cw_s = jnp.sum(ev[:, :, 2:3] * a, axis=1, keepdims=True)
        cw_e = jnp.sum(ev[:, :, 3:4] * a, axis=1, keepdims=True)
        return jax.nn.sigmoid(hw_s + cw_s), jax.nn.sigmoid(hw_e + cw_e)

    fos, foe = heads(hv_f, ev_f, a_f, 0)
    bss, bse = heads(hv_b, ev_b, a_b, 1)
    out_ref[0] = fos
    out_ref[1] = foe
    out_ref[2] = bss
    out_ref[3] = bse


def kernel(embs, h_gs, rel_embs, rel_transe_embs, params):
    p = params
    f32 = jnp.float32

    def stk(a, b):
        return jnp.stack([a, b], axis=0)

    def vpack(tag_s, tag_e, head_s, head_e):
        return jnp.concatenate(
            [p[tag_s + '_W'], p[tag_e + '_W'], p[head_s + '_W'],
             p[head_e + '_W'], p['V_W'], jnp.zeros((H, 3), f32)], axis=1)

    vecs = stk(vpack('f_start_sub_fc', 'f_end_sub_fc',
                     'f_start_obj_fc', 'f_end_obj_fc'),
               vpack('b_start_obj_fc', 'b_end_obj_fc',
                     'b_start_sub_fc', 'b_end_sub_fc'))

    def scalars(tag_s, tag_e, head_s, head_e):
        return jnp.concatenate([
            p[tag_s + '_b'], p[tag_e + '_b'], p[head_s + '_b'],
            p[head_e + '_b'], p['V_b'], jnp.zeros((3,), f32)])

    scal = stk(scalars('f_start_sub_fc', 'f_end_sub_fc',
                       'f_start_obj_fc', 'f_end_obj_fc'),
               scalars('b_start_obj_fc', 'b_end_obj_fc',
                       'b_start_sub_fc', 'b_end_sub_fc'))

    def biaspack(pre):
        rows = [p[pre + '_W_s_b'], p[pre + '_W_r_b'], p[pre + '_W_g_b'],
                p[pre + '_W_x_b'], p[pre + '_Wx2_b'],
                p['r_proj_b'], p['V_W'][:, 0], jnp.zeros((H,), f32)]
        return jnp.stack(rows, axis=0)

    bias = stk(biaspack('f'), biaspack('b'))

    whole = pl.BlockSpec(memory_space=pltpu.VMEM)
    out = pl.pallas_call(
        _brask_kernel,
        grid=(B // BT,),
        in_specs=[pl.BlockSpec((BT, L, H), lambda b: (b, 0, 0)),
                  pl.BlockSpec((BT, 1, H), lambda b: (b, 0, 0))]
        + [whole] * 15
        + [pl.BlockSpec(memory_space=pltpu.SMEM)],
        out_specs=pl.BlockSpec((4, BT, L, R), lambda b: (0, b, 0, 0)),
        out_shape=jax.ShapeDtypeStruct((4, B, L, R), f32),
        compiler_params=pltpu.CompilerParams(
            dimension_semantics=("parallel",)),
    )(embs, h_gs.reshape(B, 1, H), rel_transe_embs, rel_embs, p['r_proj_W'],
      p['f_W_s_W'], p['f_W_r_W'], p['f_W_g_W'], p['f_W_x_W'], p['f_Wx2_W'],
      p['b_W_s_W'], p['b_W_r_W'], p['b_W_g_W'], p['b_W_x_W'], p['b_Wx2_W'],
      vecs, bias, scal)

    return out
